# baseline (speedup 1.0000x reference)
import dataclasses
import functools

import numpy as np

import jax
import jax.numpy as jnp
from jax import lax
from jax.experimental import pallas as pl
from jax.experimental.pallas import tpu as pltpu
from jax.experimental.pallas import tpu_sc as plsc

N = 10000
E = 160000
D = 256
HALF = 128

NSUB = 16
CH = 40
NCH = 256
EPS = NCH * CH
EPAD = NSUB * EPS
NRING = 8
NGB = 4
BASE_ROWS = 624

_PERM = np.empty((HALF,), np.int32)
_PERM[: HALF // 2] = 2 * np.arange(HALF // 2)
_PERM[HALF // 2:] = 2 * np.arange(HALF // 2) + 1
_PERM_FULL = np.concatenate([_PERM, _PERM + HALF])

_mesh = plsc.VectorSubcoreMesh(core_axis_name="c", subcore_axis_name="s")

_sc_params = pltpu.CompilerParams()
if "needs_layout_passes" in pltpu.CompilerParams.__dataclass_fields__:
    _sc_params = dataclasses.replace(_sc_params, needs_layout_passes=False)
if "use_tc_tiling_on_sc" in pltpu.CompilerParams.__dataclass_fields__:
    _sc_params = dataclasses.replace(_sc_params, use_tc_tiling_on_sc=False)


@functools.partial(
    pl.kernel,
    out_type=jax.ShapeDtypeStruct((2, N, HALF), jnp.float32),
    mesh=_mesh,
    scratch_types=[
        pltpu.VMEM((2 * NRING, CH), jnp.int32),
        pltpu.VMEM((EPS,), jnp.float32),
        [pltpu.VMEM((CH, HALF // 2), jnp.int32)] * NGB,
        pltpu.VMEM((CH, HALF), jnp.float32),
        pltpu.VMEM((CH, HALF), jnp.float32),
        pltpu.VMEM_SHARED((N, HALF), jnp.float32),
        [pltpu.SemaphoreType.DMA] * NRING,
        [pltpu.SemaphoreType.DMA] * NGB,
        pltpu.SemaphoreType.DMA,
        pltpu.SemaphoreType.DMA,
    ],
    compiler_params=_sc_params,
)
def _spmm_sc(edges_h, vals_h, ebs_h, zeros_h, out_h,
             ering, vals_v, gbufs, s0, s1, acc,
             esems, gsems, ssem0, ssem1):
    c = lax.axis_index("c")
    s = lax.axis_index("s")

    pltpu.sync_copy(vals_h.at[s], vals_v)

    r0 = s * BASE_ROWS
    pltpu.sync_copy(zeros_h.at[pl.ds(r0, BASE_ROWS)],
                    acc.at[pl.ds(r0, BASE_ROWS)])

    @pl.when(s < 2)
    def _():
        t0 = NSUB * BASE_ROWS + s * 8
        pltpu.sync_copy(zeros_h.at[pl.ds(t0, 8)], acc.at[pl.ds(t0, 8)])

    plsc.subcore_barrier()

    ebs_c = ebs_h.at[c]

    def fetch_edges(j, slot):
        pltpu.async_copy(edges_h.at[s, j], ering.at[pl.ds(2 * slot, 2)],
                         esems[slot])

    def wait_edges(slot):
        pltpu.make_async_copy(edges_h.at[s, 0], ering.at[pl.ds(2 * slot, 2)],
                              esems[slot]).wait()

    def issue_gather(slot, b):
        pltpu.async_copy(ebs_c.at[ering.at[2 * slot]], gbufs[b], gsems[b])

    def wait_gather(b):
        pltpu.make_async_copy(ebs_c.at[ering.at[0]], gbufs[b],
                              gsems[b]).wait()

    def scale(gb, sb, base):
        @pl.loop(0, CH, step=2)
        def _(e):
            for u in range(2):
                vv = plsc.load_gather(
                    vals_v, [jnp.full((16,), base + e + u, jnp.int32)])
                src = gb.at[e + u]
                dst = sb.at[e + u]
                for k in range(HALF // 32):
                    xi = src[pl.ds(k * 16, 16)]
                    xb = plsc.bitcast(xi, jnp.bfloat16)
                    a, b = plsc.unpack(
                        xb, format=plsc.PackFormat.INTERLEAVED,
                        preferred_element_type=jnp.float32)
                    dst[pl.ds(k * 16, 16)] = a * vv
                    dst[pl.ds(HALF // 2 + k * 16, 16)] = b * vv

    for j in range(6):
        fetch_edges(j, j)
    for j in range(NGB):
        wait_edges(j)
        issue_gather(j, j)

    def visit(j, u, first):
        sb = s0 if u % 2 == 0 else s1
        ssem = ssem0 if u % 2 == 0 else ssem1
        b = u % NGB
        slot = u % NRING
        wait_gather(b)
        if not first:
            pltpu.make_async_copy(sb, acc.at[ering.at[1]], ssem).wait()
        fetch_edges(j + 6, (u + 6) % NRING)
        scale(gbufs[b], sb, j * CH)
        pltpu.async_copy(sb, acc.at[ering.at[2 * slot + 1]], ssem, add=True)
        wait_edges((u + 4) % NRING)
        issue_gather((u + 4) % NRING, b)

    for u in range(NRING):
        visit(u, u, u < 2)

    @pl.loop(NRING, NCH, step=NRING)
    def _(jj):
        for u in range(NRING):
            visit(jj + u, u, False)

    for b in range(NGB):
        wait_gather(b)
    pltpu.make_async_copy(s0, acc.at[ering.at[1]], ssem0).wait()
    pltpu.make_async_copy(s1, acc.at[ering.at[1]], ssem1).wait()
    wait_edges((NCH + 4) % NRING)
    wait_edges((NCH + 5) % NRING)
    plsc.subcore_barrier()

    out_c = out_h.at[c]
    pltpu.sync_copy(acc.at[pl.ds(r0, BASE_ROWS)],
                    out_c.at[pl.ds(r0, BASE_ROWS)])

    @pl.when(s < 2)
    def _():
        t0 = NSUB * BASE_ROWS + s * 8
        pltpu.sync_copy(acc.at[pl.ds(t0, 8)], out_c.at[pl.ds(t0, 8)])


def _tc_body(ls0_ref, ls1_ref, ebs_ref, ws_ref, wd_ref, o_ref):
    ls = jnp.concatenate([ls0_ref[0], ls1_ref[0]], axis=1)
    eb = ebs_ref[...]
    li = ls + eb
    y = jnp.dot(li, ws_ref[...], preferred_element_type=jnp.float32)
    y += jnp.dot(ls * eb, wd_ref[...], preferred_element_type=jnp.float32)
    o_ref[...] = jnp.where(y >= 0, y, 0.2 * y)


_BM = 1000


def kernel(ebs, edge_index, edge_vals, W_side, W_dot):
    rows = edge_index[0]
    cols = edge_index[1]
    pad = EPAD - E
    rows2 = jnp.pad(rows, (0, pad)).reshape(NSUB, NCH, 1, CH)
    cols2 = jnp.pad(cols, (0, pad)).reshape(NSUB, NCH, 1, CH)
    edges = jnp.concatenate([cols2, rows2], axis=2)
    edges = jnp.pad(edges, ((0, 0), (0, 6), (0, 0), (0, 0)))
    vals2 = jnp.pad(edge_vals, (0, pad)).reshape(NSUB, EPS)
    ebs_bf = ebs.astype(jnp.bfloat16)
    ebs_i32 = jnp.stack([
        lax.bitcast_convert_type(
            ebs_bf[:, h * HALF:(h + 1) * HALF].reshape(N, HALF // 2, 2),
            jnp.int32)
        for h in range(2)
    ])
    zeros = jnp.zeros((N, HALF), jnp.float32)
    ebs_perm = ebs[:, _PERM_FULL]
    ws_perm = W_side[_PERM_FULL, :]
    wd_perm = W_dot[_PERM_FULL, :]

    ls_halves = _spmm_sc(edges, vals2, ebs_i32, zeros)

    out = pl.pallas_call(
        _tc_body,
        grid=(N // _BM,),
        in_specs=[
            pl.BlockSpec((1, _BM, HALF), lambda i: (0, i, 0)),
            pl.BlockSpec((1, _BM, HALF), lambda i: (1, i, 0)),
            pl.BlockSpec((_BM, D), lambda i: (i, 0)),
            pl.BlockSpec((D, D), lambda i: (0, 0)),
            pl.BlockSpec((D, D), lambda i: (0, 0)),
        ],
        out_specs=pl.BlockSpec((_BM, D), lambda i: (i, 0)),
        out_shape=jax.ShapeDtypeStruct((N, D), jnp.float32),
    )(ls_halves, ls_halves, ebs_perm, ws_perm, wd_perm)
    return out
